# Initial kernel scaffold; baseline (speedup 1.0000x reference)
import functools

import jax
import jax.numpy as jnp
from jax import lax
from jax.experimental import pallas as pl
from jax.experimental.pallas import tpu as pltpu
from jax.experimental.pallas import tpu_sc as plsc

N_NODES = 10000
N_FEAT = 128
HID = 256
N_CLS = 64
ALPHA = 0.1

NC, NS, L = 2, 16, 16
NW = NC * NS
CHUNK = 128
ROWS_PER_TILE = N_NODES // NS

MLP_BLK = 1000


def _mlp_body(x_ref, w1_ref, b1_ref, w2_ref, b2_ref, h_ref):
    a = jnp.dot(x_ref[...], w1_ref[...], preferred_element_type=jnp.float32)
    a = jnp.maximum(a + b1_ref[...], 0.0)
    h_ref[...] = (
        jnp.dot(a, w2_ref[...], preferred_element_type=jnp.float32) + b2_ref[...]
    )


def _combine_body(h_ref, p_ref, o_ref):
    t = (1.0 + 3.0 * ALPHA) * h_ref[...] + (1.0 - ALPHA) * (p_ref[0] + p_ref[1])
    m = jnp.max(t, axis=-1, keepdims=True)
    s = t - m
    o_ref[...] = s - jnp.log(jnp.sum(jnp.exp(s), axis=-1, keepdims=True))


def _make_sc_scatter(nchunk):
    mesh = plsc.VectorSubcoreMesh(core_axis_name="c", subcore_axis_name="s")

    @functools.partial(
        pl.kernel,
        out_type=jax.ShapeDtypeStruct((NC, N_NODES, N_CLS), jnp.float32),
        mesh=mesh,
        scratch_types=[
            pltpu.VMEM_SHARED((N_NODES, N_CLS), jnp.float32),
            pltpu.VMEM((CHUNK,), jnp.int32),
            pltpu.VMEM((CHUNK,), jnp.int32),
            pltpu.VMEM((CHUNK,), jnp.float32),
            pltpu.VMEM((CHUNK, N_CLS), jnp.float32),
            pltpu.SemaphoreType.DMA,
        ],
    )
    def sc_scatter(src_hbm, dst_hbm, w_hbm, h_hbm, z_hbm, out_hbm,
                   acc_sh, idx_v, dstv, w_v, rows_v, sem):
        cid = lax.axis_index("c")
        sid = lax.axis_index("s")
        wid = sid * NC + cid

        pltpu.sync_copy(
            z_hbm.at[pl.ds(sid * ROWS_PER_TILE, ROWS_PER_TILE)],
            acc_sh.at[pl.ds(sid * ROWS_PER_TILE, ROWS_PER_TILE)],
        )
        plsc.subcore_barrier()

        ebase = wid * (CHUNK * nchunk)

        def chunk_body(g, carry):
            base = ebase + g * CHUNK
            pltpu.sync_copy(src_hbm.at[pl.ds(base, CHUNK)], idx_v)
            pltpu.sync_copy(dst_hbm.at[pl.ds(base, CHUNK)], dstv)
            pltpu.sync_copy(w_hbm.at[pl.ds(base, CHUNK)], w_v)
            pltpu.async_copy(h_hbm.at[idx_v], rows_v, sem).wait()

            def scale_body(e, c2):
                eidx = jnp.full((L,), e, jnp.int32)
                wb = plsc.load_gather(w_v, [eidx])
                for j in range(N_CLS // L):
                    cidx = lax.iota(jnp.int32, L) + (L * j)
                    r = plsc.load_gather(rows_v, [eidx, cidx])
                    plsc.store_scatter(rows_v, [eidx, cidx], r * wb)
                return c2

            lax.fori_loop(0, CHUNK, scale_body, 0)
            pltpu.sync_copy(rows_v, acc_sh.at[dstv], add=True)
            return carry

        lax.fori_loop(0, nchunk, chunk_body, 0)
        plsc.subcore_barrier()

        pltpu.sync_copy(
            acc_sh.at[pl.ds(sid * ROWS_PER_TILE, ROWS_PER_TILE)],
            out_hbm.at[cid, pl.ds(sid * ROWS_PER_TILE, ROWS_PER_TILE)],
        )

    return sc_scatter


def kernel(x, edge_index, edge_weight, W1, b1, W2, b2):
    src = edge_index[:, 0, :].astype(jnp.int32).reshape(-1)
    dst = edge_index[:, 1, :].astype(jnp.int32).reshape(-1)
    w = edge_weight.reshape(-1).astype(jnp.float32)
    e0 = src.shape[0]
    nchunk = -(-e0 // (NW * CHUNK))
    ep = NW * CHUNK * nchunk
    pad = ep - e0
    if pad:
        src = jnp.concatenate([src, jnp.zeros((pad,), jnp.int32)])
        dst = jnp.concatenate([dst, jnp.zeros((pad,), jnp.int32)])
        w = jnp.concatenate([w, jnp.zeros((pad,), jnp.float32)])

    grid = N_NODES // MLP_BLK
    h = pl.pallas_call(
        _mlp_body,
        grid=(grid,),
        in_specs=[
            pl.BlockSpec((MLP_BLK, N_FEAT), lambda i: (i, 0)),
            pl.BlockSpec((N_FEAT, HID), lambda i: (0, 0)),
            pl.BlockSpec((1, HID), lambda i: (0, 0)),
            pl.BlockSpec((HID, N_CLS), lambda i: (0, 0)),
            pl.BlockSpec((1, N_CLS), lambda i: (0, 0)),
        ],
        out_specs=pl.BlockSpec((MLP_BLK, N_CLS), lambda i: (i, 0)),
        out_shape=jax.ShapeDtypeStruct((N_NODES, N_CLS), jnp.float32),
    )(x, W1, b1.reshape(1, HID), W2, b2.reshape(1, N_CLS))

    z = jnp.zeros((N_NODES, N_CLS), jnp.float32)
    partials = _make_sc_scatter(nchunk)(src, dst, w, h, z)

    out = pl.pallas_call(
        _combine_body,
        grid=(grid,),
        in_specs=[
            pl.BlockSpec((MLP_BLK, N_CLS), lambda i: (i, 0)),
            pl.BlockSpec((NC, MLP_BLK, N_CLS), lambda i: (0, i, 0)),
        ],
        out_specs=pl.BlockSpec((MLP_BLK, N_CLS), lambda i: (i, 0)),
        out_shape=jax.ShapeDtypeStruct((N_NODES, N_CLS), jnp.float32),
    )(h, partials)
    return out


# speedup vs baseline: 4.7646x; 4.7646x over previous
import functools

import jax
import jax.numpy as jnp
from jax import lax
from jax.experimental import pallas as pl
from jax.experimental.pallas import tpu as pltpu
from jax.experimental.pallas import tpu_sc as plsc

N_NODES = 10000
N_FEAT = 128
HID = 256
N_CLS = 64
ALPHA = 0.1

NC, NS, L = 2, 16, 16
NW = NC * NS
CHUNK = 128
N_PAD = 10240
ROWS_PER_TILE = N_PAD // NS

MLP_BLK = 1024


def _mlp_body(x_ref, w1_ref, b1_ref, w2_ref, b2_ref, h_ref):
    a = jnp.dot(x_ref[...], w1_ref[...], preferred_element_type=jnp.float32)
    a = jnp.maximum(a + b1_ref[...], 0.0)
    h_ref[...] = (
        jnp.dot(a, w2_ref[...], preferred_element_type=jnp.float32) + b2_ref[...]
    )


def _combine_body(h_ref, p_ref, o_ref):
    t = (1.0 + 3.0 * ALPHA) * h_ref[...] + (1.0 - ALPHA) * (p_ref[0] + p_ref[1])
    m = jnp.max(t, axis=-1, keepdims=True)
    s = t - m
    o_ref[...] = s - jnp.log(jnp.sum(jnp.exp(s), axis=-1, keepdims=True))


def _make_sc_scatter(nchunk):
    mesh = plsc.VectorSubcoreMesh(core_axis_name="c", subcore_axis_name="s")

    @functools.partial(
        pl.kernel,
        out_type=jax.ShapeDtypeStruct((NC, N_PAD, N_CLS), jnp.float32),
        mesh=mesh,
        compiler_params=pltpu.CompilerParams(use_tc_tiling_on_sc=False),
        scratch_types=[
            pltpu.VMEM_SHARED((N_PAD, N_CLS), jnp.float32),
            pltpu.VMEM((CHUNK,), jnp.int32),
            pltpu.VMEM((CHUNK,), jnp.int32),
            pltpu.VMEM((CHUNK,), jnp.float32),
            pltpu.VMEM((CHUNK, N_CLS), jnp.float32),
            pltpu.SemaphoreType.DMA,
        ],
    )
    def sc_scatter(src_hbm, dst_hbm, w_hbm, h_hbm, z_hbm, out_hbm,
                   acc_sh, idx_v, dstv, w_v, rows_v, sem):
        cid = lax.axis_index("c")
        sid = lax.axis_index("s")
        wid = sid * NC + cid

        pltpu.sync_copy(
            z_hbm.at[pl.ds(sid * ROWS_PER_TILE, ROWS_PER_TILE)],
            acc_sh.at[pl.ds(sid * ROWS_PER_TILE, ROWS_PER_TILE)],
        )
        plsc.subcore_barrier()

        ebase = wid * (CHUNK * nchunk)

        def chunk_body(g, carry):
            base = ebase + g * CHUNK
            pltpu.sync_copy(src_hbm.at[pl.ds(base, CHUNK)], idx_v)
            pltpu.sync_copy(dst_hbm.at[pl.ds(base, CHUNK)], dstv)
            pltpu.sync_copy(w_hbm.at[pl.ds(base, CHUNK)], w_v)
            pltpu.async_copy(h_hbm.at[idx_v], rows_v, sem).wait()

            for g2 in range(CHUNK // L):
                w16 = w_v[pl.ds(g2 * L, L)]
                for i in range(L):
                    e = g2 * L + i
                    wb = jnp.full(
                        (L,), lax.squeeze(lax.slice(w16, (i,), (i + 1,)), (0,))
                    )
                    for j in range(N_CLS // L):
                        rows_v[e, pl.ds(j * L, L)] = (
                            rows_v[e, pl.ds(j * L, L)] * wb
                        )
            pltpu.sync_copy(rows_v, acc_sh.at[dstv], add=True)
            return carry

        lax.fori_loop(0, nchunk, chunk_body, 0)
        plsc.subcore_barrier()

        pltpu.sync_copy(
            acc_sh.at[pl.ds(sid * ROWS_PER_TILE, ROWS_PER_TILE)],
            out_hbm.at[cid, pl.ds(sid * ROWS_PER_TILE, ROWS_PER_TILE)],
        )

    return sc_scatter


def kernel(x, edge_index, edge_weight, W1, b1, W2, b2):
    src = edge_index[:, 0, :].astype(jnp.int32).reshape(-1)
    dst = edge_index[:, 1, :].astype(jnp.int32).reshape(-1)
    w = edge_weight.reshape(-1).astype(jnp.float32)
    e0 = src.shape[0]
    nchunk = -(-e0 // (NW * CHUNK))
    ep = NW * CHUNK * nchunk
    pad = ep - e0
    if pad:
        src = jnp.concatenate([src, jnp.zeros((pad,), jnp.int32)])
        dst = jnp.concatenate([dst, jnp.zeros((pad,), jnp.int32)])
        w = jnp.concatenate([w, jnp.zeros((pad,), jnp.float32)])

    xp = jnp.pad(x, ((0, N_PAD - N_NODES), (0, 0)))
    grid = N_PAD // MLP_BLK
    h = pl.pallas_call(
        _mlp_body,
        grid=(grid,),
        in_specs=[
            pl.BlockSpec((MLP_BLK, N_FEAT), lambda i: (i, 0)),
            pl.BlockSpec((N_FEAT, HID), lambda i: (0, 0)),
            pl.BlockSpec((1, HID), lambda i: (0, 0)),
            pl.BlockSpec((HID, N_CLS), lambda i: (0, 0)),
            pl.BlockSpec((1, N_CLS), lambda i: (0, 0)),
        ],
        out_specs=pl.BlockSpec((MLP_BLK, N_CLS), lambda i: (i, 0)),
        out_shape=jax.ShapeDtypeStruct((N_PAD, N_CLS), jnp.float32),
    )(xp, W1, b1.reshape(1, HID), W2, b2.reshape(1, N_CLS))

    z = jnp.zeros((N_PAD, N_CLS), jnp.float32)
    partials = _make_sc_scatter(nchunk)(src, dst, w, h, z)

    cgrid = N_NODES // 1000
    out = pl.pallas_call(
        _combine_body,
        grid=(cgrid,),
        in_specs=[
            pl.BlockSpec((1000, N_CLS), lambda i: (i, 0)),
            pl.BlockSpec((NC, 1000, N_CLS), lambda i: (0, i, 0)),
        ],
        out_specs=pl.BlockSpec((1000, N_CLS), lambda i: (i, 0)),
        out_shape=jax.ShapeDtypeStruct((N_NODES, N_CLS), jnp.float32),
    )(h, partials)
    return out
